# baseline (speedup 1.0000x reference)
import jax
import jax.numpy as jnp
from jax.experimental import pallas as pl
from jax.experimental.pallas import tpu as pltpu

B, N, F_IN, D, C = 64, 128, 128, 128, 2


def _gnn_body(f1_ref, a1_ref, f2_ref, a2_ref, s1_ref, s2_ref,
              W1_ref, b1_ref, W2_ref, b2_ref, Wa_ref, ba_ref, Wc_ref, bc_ref,
              out_ref):
    iota = jax.lax.broadcasted_iota(jnp.int32, (N, 1), 0).astype(jnp.float32)

    def emb(f_ref, a_ref, size):
        m = (iota < size).astype(jnp.float32)
        x = f_ref[0] * m
        A = a_ref[0] * m * m.T
        h = jnp.dot(x, W1_ref[...], preferred_element_type=jnp.float32) + b1_ref[...]
        h = jnp.dot(A, h, preferred_element_type=jnp.float32)
        h = jnp.maximum(h, 0.0)
        h = jnp.dot(h, W2_ref[...], preferred_element_type=jnp.float32) + b2_ref[...]
        h = jnp.dot(A, h, preferred_element_type=jnp.float32)
        h = jnp.maximum(h, 0.0) * m
        g = (jnp.dot(h, Wa_ref[...], preferred_element_type=jnp.float32) + ba_ref[...]) * m
        return jnp.sum(g, axis=0, keepdims=True) / jnp.maximum(size, 1.0)

    e1 = emb(f1_ref, a1_ref, s1_ref[0, 0, 0])
    e2 = emb(f2_ref, a2_ref, s2_ref[0, 0, 0])
    out_ref[0] = (jnp.dot(e1, Wc_ref[:D], preferred_element_type=jnp.float32)
                  + jnp.dot(e2, Wc_ref[D:], preferred_element_type=jnp.float32)
                  + bc_ref[...])


def kernel(feats_1, adjs_1, feats_2, adjs_2, sizes_1, sizes_2,
           W1, b1, W2, b2, Wa, ba, Wc, bc):
    s1 = sizes_1.astype(jnp.float32).reshape(B, 1, 1)
    s2 = sizes_2.astype(jnp.float32).reshape(B, 1, 1)
    b1r = b1.reshape(1, D)
    b2r = b2.reshape(1, D)
    bar = ba.reshape(1, D)
    bcr = bc.reshape(1, C)

    per_graph = lambda b: (b, 0, 0)
    per_row = lambda b: (b, 0)
    fixed = lambda b: (0, 0)

    out = pl.pallas_call(
        _gnn_body,
        grid=(B,),
        in_specs=[
            pl.BlockSpec((1, N, F_IN), per_graph),
            pl.BlockSpec((1, N, N), per_graph),
            pl.BlockSpec((1, N, F_IN), per_graph),
            pl.BlockSpec((1, N, N), per_graph),
            pl.BlockSpec((1, 1, 1), per_graph),
            pl.BlockSpec((1, 1, 1), per_graph),
            pl.BlockSpec((F_IN, D), fixed),
            pl.BlockSpec((1, D), fixed),
            pl.BlockSpec((D, D), fixed),
            pl.BlockSpec((1, D), fixed),
            pl.BlockSpec((D, D), fixed),
            pl.BlockSpec((1, D), fixed),
            pl.BlockSpec((2 * D, C), fixed),
            pl.BlockSpec((1, C), fixed),
        ],
        out_specs=pl.BlockSpec((1, 1, C), per_graph),
        out_shape=jax.ShapeDtypeStruct((B, 1, C), jnp.float32),
        compiler_params=pltpu.CompilerParams(
            dimension_semantics=("arbitrary",),
        ),
    )(feats_1, adjs_1, feats_2, adjs_2, s1, s2,
      W1, b1r, W2, b2r, Wa, bar, Wc, bcr)
    return out.reshape(B, C)


# speedup vs baseline: 3.5126x; 3.5126x over previous
import jax
import jax.numpy as jnp
from jax.experimental import pallas as pl
from jax.experimental.pallas import tpu as pltpu

B, N, F_IN, D, C = 64, 128, 128, 128, 2
BG = 8


def _gnn_body(f1_ref, a1_ref, f2_ref, a2_ref, s1_ref, s2_ref,
              W1_ref, b1_ref, W2_ref, b2_ref, Wa_ref, ba_ref, Wc_ref, bc_ref,
              out_ref):
    iota_n = jax.lax.broadcasted_iota(jnp.int32, (1, N), 1).astype(jnp.float32)

    seg_r = jax.lax.broadcasted_iota(jnp.int32, (BG, BG * N), 0)
    seg_c = jax.lax.broadcasted_iota(jnp.int32, (BG, BG * N), 1)

    def emb(f_ref, a_ref, s_ref):
        sizes = s_ref[...]
        x = f_ref[...].reshape(BG * N, F_IN)
        h = jnp.dot(x, W1_ref[...], preferred_element_type=jnp.float32) + b1_ref[...]
        As = []
        parts = []
        for g in range(BG):
            cm = (iota_n < sizes[g, 0]).astype(jnp.float32)
            A_g = a_ref[g] * cm
            As.append(A_g)
            t = jnp.dot(A_g, h[g * N:(g + 1) * N], preferred_element_type=jnp.float32)
            parts.append(jnp.maximum(t, 0.0))
        h = jnp.concatenate(parts, axis=0)
        h = jnp.dot(h, W2_ref[...], preferred_element_type=jnp.float32) + b2_ref[...]
        parts = []
        for g in range(BG):
            t = jnp.dot(As[g], h[g * N:(g + 1) * N], preferred_element_type=jnp.float32)
            parts.append(jnp.maximum(t, 0.0))
        h = jnp.concatenate(parts, axis=0)
        g_all = jnp.dot(h, Wa_ref[...], preferred_element_type=jnp.float32) + ba_ref[...]
        M = ((seg_c // N == seg_r) & ((seg_c % N).astype(jnp.float32) < sizes)
             ).astype(jnp.float32)
        sums = jnp.dot(M, g_all, preferred_element_type=jnp.float32)
        return sums / jnp.maximum(sizes, 1.0)

    e1 = emb(f1_ref, a1_ref, s1_ref)
    e2 = emb(f2_ref, a2_ref, s2_ref)
    out_ref[...] = (jnp.dot(e1, Wc_ref[:D], preferred_element_type=jnp.float32)
                    + jnp.dot(e2, Wc_ref[D:], preferred_element_type=jnp.float32)
                    + bc_ref[...])


def kernel(feats_1, adjs_1, feats_2, adjs_2, sizes_1, sizes_2,
           W1, b1, W2, b2, Wa, ba, Wc, bc):
    s1 = sizes_1.astype(jnp.float32).reshape(B, 1)
    s2 = sizes_2.astype(jnp.float32).reshape(B, 1)
    b1r = b1.reshape(1, D)
    b2r = b2.reshape(1, D)
    bar = ba.reshape(1, D)
    bcr = bc.reshape(1, C)

    per_graph = lambda b: (b, 0, 0)
    per_row = lambda b: (b, 0)
    fixed = lambda b: (0, 0)

    out = pl.pallas_call(
        _gnn_body,
        grid=(B // BG,),
        in_specs=[
            pl.BlockSpec((BG, N, F_IN), per_graph),
            pl.BlockSpec((BG, N, N), per_graph),
            pl.BlockSpec((BG, N, F_IN), per_graph),
            pl.BlockSpec((BG, N, N), per_graph),
            pl.BlockSpec((BG, 1), per_row),
            pl.BlockSpec((BG, 1), per_row),
            pl.BlockSpec((F_IN, D), fixed),
            pl.BlockSpec((1, D), fixed),
            pl.BlockSpec((D, D), fixed),
            pl.BlockSpec((1, D), fixed),
            pl.BlockSpec((D, D), fixed),
            pl.BlockSpec((1, D), fixed),
            pl.BlockSpec((2 * D, C), fixed),
            pl.BlockSpec((1, C), fixed),
        ],
        out_specs=pl.BlockSpec((BG, C), per_row),
        out_shape=jax.ShapeDtypeStruct((B, C), jnp.float32),
        compiler_params=pltpu.CompilerParams(
            dimension_semantics=("arbitrary",),
        ),
    )(feats_1, adjs_1, feats_2, adjs_2, s1, s2,
      W1, b1r, W2, b2r, Wa, bar, Wc, bcr)
    return out
